# baseline (speedup 1.0000x reference)
import functools

import jax
import jax.numpy as jnp
from jax import lax
from jax.experimental import pallas as pl
from jax.experimental.pallas import tpu as pltpu
from jax.experimental.pallas import tpu_sc as plsc

_N = 10000
_E = 320000
_D = 128
_DH = 100

_NB = 128
_NC = 2
_NS = 16
_NW = _NC * _NS
_T = 80
_TS = 160
_EPAD = _NS * _TS * _NB
_DH2 = 64
_NPAD = 10112
_RPT = _NPAD // _NS

_RB = 1000
_GRID = _N // _RB

_BUF = 5
_ALD = 3
_HBM_POS = (0,)

_sc_mesh = plsc.VectorSubcoreMesh(core_axis_name="c", subcore_axis_name="s")


@functools.partial(
    pl.kernel,
    out_type=(
        jax.ShapeDtypeStruct((_NC, _NPAD, 16), jnp.float32),
        jax.ShapeDtypeStruct((_NC, _NPAD, 16), jnp.float32),
    ),
    mesh=_sc_mesh,
    scratch_types=[
        pltpu.VMEM((_T, 2, _NB), jnp.int32),
        pltpu.VMEM((_NB, 16), jnp.float32),
        pltpu.VMEM_SHARED((_NPAD, 16), jnp.float32),
        pltpu.VMEM_SHARED((_NPAD, 16), jnp.float32),
        pltpu.SemaphoreType.DMA,
    ],
)
def _deg_kernel(srcdst, ones_hbm, z16_hbm, dpo, dpi,
                idx_v, ones_v, sh_o, sh_i, sem):
    cid = lax.axis_index("c")
    sid = lax.axis_index("s")
    r0 = sid * _RPT
    pltpu.sync_copy(z16_hbm.at[pl.ds(r0, _RPT)], sh_o.at[pl.ds(r0, _RPT)])
    pltpu.sync_copy(z16_hbm.at[pl.ds(r0, _RPT)], sh_i.at[pl.ds(r0, _RPT)])
    pltpu.sync_copy(ones_hbm, ones_v)
    pltpu.sync_copy(srcdst.at[sid, pl.ds(cid * _T, _T)], idx_v)
    plsc.subcore_barrier()

    def body(j, carry):
        pltpu.async_copy(ones_v, sh_o.at[idx_v.at[j, 0]], sem, add=True)
        pltpu.async_copy(ones_v, sh_i.at[idx_v.at[j, 1]], sem, add=True)
        return carry

    lax.fori_loop(0, _T, body, 0)

    def drain(j, carry):
        pltpu.make_async_copy(ones_v, sh_o.at[idx_v.at[0, 0]], sem).wait()
        return carry

    lax.fori_loop(0, 2 * _T, drain, 0)
    plsc.subcore_barrier()
    pltpu.sync_copy(sh_o.at[pl.ds(r0, _RPT)], dpo.at[cid, pl.ds(r0, _RPT)])
    pltpu.sync_copy(sh_i.at[pl.ds(r0, _RPT)], dpi.at[cid, pl.ds(r0, _RPT)])


@functools.partial(
    pl.kernel,
    out_type=jax.ShapeDtypeStruct((_NC, _NPAD, _DH2), jnp.float32),
    mesh=_sc_mesh,
    scratch_types=[
        pltpu.VMEM((_BUF, 2, _NB), jnp.int32),
        pltpu.VMEM((_BUF, _NB, _DH2), jnp.float32),
        pltpu.VMEM_SHARED((_NPAD, _DH2), jnp.float32),
        pltpu.VMEM_SHARED((_NPAD, _DH2), jnp.float32),
    ] + [pltpu.SemaphoreType.DMA] * _BUF,
    compiler_params=pltpu.CompilerParams(use_tc_tiling_on_sc=False),
)
def _msg_kernel(h_hbm, srcdst, z64_hbm, out_hbm,
                idx_v, rows_v, agg_sh, h_sh, *sems):
    cid = lax.axis_index("c")
    sid = lax.axis_index("s")
    r0 = sid * _RPT
    pltpu.sync_copy(z64_hbm.at[pl.ds(r0, _RPT)], agg_sh.at[pl.ds(r0, _RPT)])
    nrow = _N // _NS
    h0 = sid * nrow
    pltpu.sync_copy(h_hbm.at[cid, pl.ds(h0, nrow)], h_sh.at[pl.ds(h0, nrow)])
    hh = h_sh
    hbm_h = h_hbm.at[cid]
    sd = srcdst.at[sid]
    plsc.subcore_barrier()

    def i_issue(j, b):
        pltpu.async_copy(sd.at[j], idx_v.at[b], sems[b])

    def i_wait(j, b):
        pltpu.make_async_copy(sd.at[j], idx_v.at[b], sems[b]).wait()

    def g_issue(j, b):
        src_tab = hbm_h if b in _HBM_POS else hh
        pltpu.async_copy(src_tab.at[idx_v.at[b, 0]], rows_v.at[b], sems[b])

    def g_wait(j, b):
        src_tab = hbm_h if b in _HBM_POS else hh
        pltpu.make_async_copy(src_tab.at[idx_v.at[b, 0]], rows_v.at[b],
                              sems[b]).wait()

    def s_issue(j, b):
        pltpu.async_copy(rows_v.at[b], agg_sh.at[idx_v.at[b, 1]], sems[b],
                         add=True)

    def s_wait(j, b):
        pltpu.make_async_copy(rows_v.at[b], agg_sh.at[idx_v.at[b, 1]],
                              sems[b]).wait()

    for j in range(_ALD + 1):
        i_issue(j, j)
    for j in range(_ALD):
        i_wait(j, j)
        g_issue(j, j)

    def body(k, carry):
        for b in range(_BUF):
            j = k * _BUF + b
            g_wait(j, b)
            s_issue(j, b)
            b2 = (b + _ALD + 1) % _BUF

            @pl.when(j + _ALD + 1 - _BUF >= 0)
            def _():
                s_wait(j + _ALD + 1 - _BUF, b2)

            @pl.when(j + _ALD + 1 < _TS)
            def _():
                i_issue(j + _ALD + 1, b2)

            b1 = (b + _ALD) % _BUF

            @pl.when(j + _ALD < _TS)
            def _():
                i_wait(j + _ALD, b1)
                g_issue(j + _ALD, b1)
        return carry

    lax.fori_loop(0, _TS // _BUF, body, 0)
    for j in range(_TS - (_BUF - _ALD - 1), _TS):
        s_wait(j, j % _BUF)
    plsc.subcore_barrier()
    pltpu.sync_copy(agg_sh.at[pl.ds(r0, _RPT)], out_hbm.at[cid, pl.ds(r0, _RPT)])


def _norm_from(deg):
    return jnp.where(deg > 0.0, lax.rsqrt(jnp.maximum(deg, 1.0)), 0.0)


def _swish(v):
    return v * jax.nn.sigmoid(v)


def _tc_pre_body(x_ref, w_ref, d_ref, o_ref):
    deg = d_ref[0, :, 0:1] + d_ref[1, :, 0:1]
    h = jnp.dot(x_ref[...], w_ref[...], preferred_element_type=jnp.float32)
    h = h * _norm_from(deg)
    o_ref[0] = h[:, :_DH2]
    o_ref[1] = h[:, _DH2:]


def _tc_mid_body(p_ref, di_ref, b_ref, w_ref, do_ref, o_ref):
    deg_in = di_ref[0, :, 0:1] + di_ref[1, :, 0:1]
    agg = jnp.concatenate([p_ref[0], p_ref[1]], axis=-1) * _norm_from(deg_in)
    h = _swish(agg + b_ref[...])
    deg_out = do_ref[0, :, 0:1] + do_ref[1, :, 0:1]
    h = jnp.dot(h, w_ref[...],
                preferred_element_type=jnp.float32) * _norm_from(deg_out)
    o_ref[0] = h[:, :_DH2]
    o_ref[1] = h[:, _DH2:]


def _tc_head_body(p_ref, di_ref, b_ref, wd_ref, bd_ref, wo_ref, bo_ref, o_ref):
    deg_in = di_ref[0, :, 0:1] + di_ref[1, :, 0:1]
    agg = jnp.concatenate([p_ref[0], p_ref[1]], axis=-1) * _norm_from(deg_in)
    h = _swish(agg + b_ref[...])
    d = _swish(jnp.dot(h, wd_ref[...], preferred_element_type=jnp.float32)
               + bd_ref[...])
    logit = jnp.dot(d, wo_ref[...], preferred_element_type=jnp.float32) \
        + bo_ref[...]
    o_ref[...] = jax.nn.sigmoid(logit)


def _row_spec(cols):
    return pl.BlockSpec((_RB, cols), lambda i: (i, 0))


def _pair_spec(cols):
    return pl.BlockSpec((2, _RB, cols), lambda i: (0, i, 0))


def _full_spec(rows, cols):
    return pl.BlockSpec((rows, cols), lambda i: (0, 0))


def kernel(x, edge_index, W1, b1, W2, b2, Wd, bd, Wo, bo):
    src = edge_index[0]
    dst = edge_index[1]
    pad_e = _EPAD - _E
    srcw = jnp.concatenate(
        [src, jnp.zeros((pad_e,), jnp.int32)]).reshape(_NS, _TS, _NB)
    dstw = jnp.concatenate(
        [dst, jnp.full((pad_e,), _N, jnp.int32)]).reshape(_NS, _TS, _NB)
    srcdst = jnp.stack([srcw, dstw], axis=2)
    ones2d = jnp.ones((_NB, 16), jnp.float32)
    z16 = jnp.zeros((_NPAD, 16), jnp.float32)
    z64 = jnp.zeros((_NPAD, _DH2), jnp.float32)

    dpo, dpi = _deg_kernel(srcdst, ones2d, z16)

    b1r = b1.reshape(1, -1)
    b2r = b2.reshape(1, -1)
    wd_p = jnp.zeros((_D, _D), jnp.float32).at[:, :_DH].set(Wd)
    bd_p = jnp.zeros((1, _D), jnp.float32).at[0, :_DH].set(bd)
    wo_p = jnp.zeros((_D, 1), jnp.float32).at[:_DH, :].set(Wo)
    bo_p = bo.reshape(1, 1)

    h1s = pl.pallas_call(
        _tc_pre_body,
        grid=(_GRID,),
        in_specs=[_row_spec(_D), _full_spec(_D, _D), _pair_spec(16)],
        out_specs=_pair_spec(_DH2),
        out_shape=jax.ShapeDtypeStruct((_NC, _N, _DH2), jnp.float32),
    )(x, W1, dpo)

    p1 = _msg_kernel(h1s, srcdst, z64)

    h2s = pl.pallas_call(
        _tc_mid_body,
        grid=(_GRID,),
        in_specs=[_pair_spec(_DH2), _pair_spec(16), _full_spec(1, _D),
                  _full_spec(_D, _D), _pair_spec(16)],
        out_specs=_pair_spec(_DH2),
        out_shape=jax.ShapeDtypeStruct((_NC, _N, _DH2), jnp.float32),
    )(p1, dpi, b1r, W2, dpo)

    p2 = _msg_kernel(h2s, srcdst, z64)

    out = pl.pallas_call(
        _tc_head_body,
        grid=(_GRID,),
        in_specs=[_pair_spec(_DH2), _pair_spec(16), _full_spec(1, _D),
                  _full_spec(_D, _D), _full_spec(1, _D),
                  _full_spec(_D, 1), _full_spec(1, 1)],
        out_specs=pl.BlockSpec((_RB, 1), lambda i: (i, 0)),
        out_shape=jax.ShapeDtypeStruct((_N, 1), jnp.float32),
    )(p2, dpi, b2r, wd_p, bd_p, wo_p, bo_p)

    return out


# speedup vs baseline: 1.2294x; 1.2294x over previous
import functools

import jax
import jax.numpy as jnp
from jax import lax
from jax.experimental import pallas as pl
from jax.experimental.pallas import tpu as pltpu
from jax.experimental.pallas import tpu_sc as plsc

_N = 10000
_E = 320000
_D = 128
_DH = 100

_NB = 128
_NC = 2
_NS = 16
_NW = _NC * _NS
_T = 80
_TS = 160
_EPAD = _NS * _TS * _NB
_DH2 = 64
_NPAD = 10112
_RPT = _NPAD // _NS

_RB = 1000
_GRID = _N // _RB

_BUF = 5
_ALD = 3
_HBM_POS = ()

_sc_mesh = plsc.VectorSubcoreMesh(core_axis_name="c", subcore_axis_name="s")


@functools.partial(
    pl.kernel,
    out_type=(
        jax.ShapeDtypeStruct((_NC, _NPAD, 16), jnp.float32),
        jax.ShapeDtypeStruct((_NC, _NPAD, 16), jnp.float32),
    ),
    mesh=_sc_mesh,
    scratch_types=[
        pltpu.VMEM((_T, 2, _NB), jnp.int32),
        pltpu.VMEM((_NB, 16), jnp.float32),
        pltpu.VMEM_SHARED((_NPAD, 16), jnp.float32),
        pltpu.VMEM_SHARED((_NPAD, 16), jnp.float32),
        pltpu.SemaphoreType.DMA,
    ],
)
def _deg_kernel(srcdst, ones_hbm, z16_hbm, dpo, dpi,
                idx_v, ones_v, sh_o, sh_i, sem):
    cid = lax.axis_index("c")
    sid = lax.axis_index("s")
    r0 = sid * _RPT
    pltpu.sync_copy(z16_hbm.at[pl.ds(r0, _RPT)], sh_o.at[pl.ds(r0, _RPT)])
    pltpu.sync_copy(z16_hbm.at[pl.ds(r0, _RPT)], sh_i.at[pl.ds(r0, _RPT)])
    pltpu.sync_copy(ones_hbm, ones_v)
    pltpu.sync_copy(srcdst.at[sid, pl.ds(cid * _T, _T)], idx_v)
    plsc.subcore_barrier()

    def body(j, carry):
        pltpu.async_copy(ones_v, sh_o.at[idx_v.at[j, 0]], sem, add=True)
        pltpu.async_copy(ones_v, sh_i.at[idx_v.at[j, 1]], sem, add=True)
        return carry

    lax.fori_loop(0, _T, body, 0)

    def drain(j, carry):
        pltpu.make_async_copy(ones_v, sh_o.at[idx_v.at[0, 0]], sem).wait()
        return carry

    lax.fori_loop(0, 2 * _T, drain, 0)
    plsc.subcore_barrier()
    pltpu.sync_copy(sh_o.at[pl.ds(r0, _RPT)], dpo.at[cid, pl.ds(r0, _RPT)])
    pltpu.sync_copy(sh_i.at[pl.ds(r0, _RPT)], dpi.at[cid, pl.ds(r0, _RPT)])


@functools.partial(
    pl.kernel,
    out_type=jax.ShapeDtypeStruct((_NC, _NPAD, _DH2), jnp.float32),
    mesh=_sc_mesh,
    scratch_types=[
        pltpu.VMEM((_BUF, 2, _NB), jnp.int32),
        pltpu.VMEM((_BUF, _NB, _DH2), jnp.float32),
        pltpu.VMEM_SHARED((_NPAD, _DH2), jnp.float32),
        pltpu.VMEM_SHARED((_NPAD, _DH2), jnp.float32),
    ] + [pltpu.SemaphoreType.DMA] * _BUF,
    compiler_params=pltpu.CompilerParams(use_tc_tiling_on_sc=False),
)
def _msg_kernel(h_hbm, srcdst, z64_hbm, out_hbm,
                idx_v, rows_v, agg_sh, h_sh, *sems):
    cid = lax.axis_index("c")
    sid = lax.axis_index("s")
    r0 = sid * _RPT
    pltpu.sync_copy(z64_hbm.at[pl.ds(r0, _RPT)], agg_sh.at[pl.ds(r0, _RPT)])
    nrow = _N // _NS
    h0 = sid * nrow
    pltpu.sync_copy(h_hbm.at[cid, pl.ds(h0, nrow)], h_sh.at[pl.ds(h0, nrow)])
    hh = h_sh
    hbm_h = h_hbm.at[cid]
    sd = srcdst.at[sid]
    plsc.subcore_barrier()

    def i_issue(j, b):
        pltpu.async_copy(sd.at[j], idx_v.at[b], sems[b])

    def i_wait(j, b):
        pltpu.make_async_copy(sd.at[j], idx_v.at[b], sems[b]).wait()

    def g_issue(j, b):
        src_tab = hbm_h if b in _HBM_POS else hh
        pltpu.async_copy(src_tab.at[idx_v.at[b, 0]], rows_v.at[b], sems[b])

    def g_wait(j, b):
        src_tab = hbm_h if b in _HBM_POS else hh
        pltpu.make_async_copy(src_tab.at[idx_v.at[b, 0]], rows_v.at[b],
                              sems[b]).wait()

    def s_issue(j, b):
        pltpu.async_copy(rows_v.at[b], agg_sh.at[idx_v.at[b, 1]], sems[b],
                         add=True)

    def s_wait(j, b):
        pltpu.make_async_copy(rows_v.at[b], agg_sh.at[idx_v.at[b, 1]],
                              sems[b]).wait()

    for j in range(_ALD + 1):
        i_issue(j, j)
    for j in range(_ALD):
        i_wait(j, j)
        g_issue(j, j)

    def body(k, carry):
        for b in range(_BUF):
            j = k * _BUF + b
            g_wait(j, b)
            s_issue(j, b)
            b2 = (b + _ALD + 1) % _BUF

            @pl.when(j + _ALD + 1 - _BUF >= 0)
            def _():
                s_wait(j + _ALD + 1 - _BUF, b2)

            @pl.when(j + _ALD + 1 < _TS)
            def _():
                i_issue(j + _ALD + 1, b2)

            b1 = (b + _ALD) % _BUF

            @pl.when(j + _ALD < _TS)
            def _():
                i_wait(j + _ALD, b1)
                g_issue(j + _ALD, b1)
        return carry

    lax.fori_loop(0, _TS // _BUF, body, 0)
    for j in range(_TS - (_BUF - _ALD - 1), _TS):
        s_wait(j, j % _BUF)
    plsc.subcore_barrier()
    pltpu.sync_copy(agg_sh.at[pl.ds(r0, _RPT)], out_hbm.at[cid, pl.ds(r0, _RPT)])


def _norm_from(deg):
    return jnp.where(deg > 0.0, lax.rsqrt(jnp.maximum(deg, 1.0)), 0.0)


def _swish(v):
    return v * jax.nn.sigmoid(v)


def _tc_mm_body(x_ref, w_ref, o_ref):
    o_ref[...] = jnp.dot(x_ref[...], w_ref[...],
                         preferred_element_type=jnp.float32)


def _tc_scale_body(h_ref, d_ref, o_ref):
    deg = d_ref[0, :, 0:1] + d_ref[1, :, 0:1]
    h = h_ref[...] * _norm_from(deg)
    o_ref[0] = h[:, :_DH2]
    o_ref[1] = h[:, _DH2:]


def _tc_mid_body(p_ref, di_ref, b_ref, w_ref, do_ref, o_ref):
    deg_in = di_ref[0, :, 0:1] + di_ref[1, :, 0:1]
    agg = jnp.concatenate([p_ref[0], p_ref[1]], axis=-1) * _norm_from(deg_in)
    h = _swish(agg + b_ref[...])
    deg_out = do_ref[0, :, 0:1] + do_ref[1, :, 0:1]
    h = jnp.dot(h, w_ref[...],
                preferred_element_type=jnp.float32) * _norm_from(deg_out)
    o_ref[0] = h[:, :_DH2]
    o_ref[1] = h[:, _DH2:]


def _tc_head_body(p_ref, di_ref, b_ref, wd_ref, bd_ref, wo_ref, bo_ref, o_ref):
    deg_in = di_ref[0, :, 0:1] + di_ref[1, :, 0:1]
    agg = jnp.concatenate([p_ref[0], p_ref[1]], axis=-1) * _norm_from(deg_in)
    h = _swish(agg + b_ref[...])
    d = _swish(jnp.dot(h, wd_ref[...], preferred_element_type=jnp.float32)
               + bd_ref[...])
    logit = jnp.dot(d, wo_ref[...], preferred_element_type=jnp.float32) \
        + bo_ref[...]
    o_ref[...] = jax.nn.sigmoid(logit)


def _row_spec(cols):
    return pl.BlockSpec((_RB, cols), lambda i: (i, 0))


def _pair_spec(cols):
    return pl.BlockSpec((2, _RB, cols), lambda i: (0, i, 0))


def _full_spec(rows, cols):
    return pl.BlockSpec((rows, cols), lambda i: (0, 0))


def kernel(x, edge_index, W1, b1, W2, b2, Wd, bd, Wo, bo):
    src = edge_index[0]
    dst = edge_index[1]
    pad_e = _EPAD - _E
    srcw = jnp.concatenate(
        [src, jnp.zeros((pad_e,), jnp.int32)]).reshape(_NS, _TS, _NB)
    dstw = jnp.concatenate(
        [dst, jnp.full((pad_e,), _N, jnp.int32)]).reshape(_NS, _TS, _NB)
    srcdst = jnp.stack([srcw, dstw], axis=2)
    ones2d = jnp.ones((_NB, 16), jnp.float32)
    z16 = jnp.zeros((_NPAD, 16), jnp.float32)
    z64 = jnp.zeros((_NPAD, _DH2), jnp.float32)

    dpo, dpi = _deg_kernel(srcdst, ones2d, z16)

    b1r = b1.reshape(1, -1)
    b2r = b2.reshape(1, -1)
    wd_p = jnp.zeros((_D, _D), jnp.float32).at[:, :_DH].set(Wd)
    bd_p = jnp.zeros((1, _D), jnp.float32).at[0, :_DH].set(bd)
    wo_p = jnp.zeros((_D, 1), jnp.float32).at[:_DH, :].set(Wo)
    bo_p = bo.reshape(1, 1)

    xw = pl.pallas_call(
        _tc_mm_body,
        grid=(_GRID,),
        in_specs=[_row_spec(_D), _full_spec(_D, _D)],
        out_specs=_row_spec(_D),
        out_shape=jax.ShapeDtypeStruct((_N, _D), jnp.float32),
    )(x, W1)

    h1s = pl.pallas_call(
        _tc_scale_body,
        grid=(_GRID,),
        in_specs=[_row_spec(_D), _pair_spec(16)],
        out_specs=_pair_spec(_DH2),
        out_shape=jax.ShapeDtypeStruct((_NC, _N, _DH2), jnp.float32),
    )(xw, dpo)

    p1 = _msg_kernel(h1s, srcdst, z64)

    h2s = pl.pallas_call(
        _tc_mid_body,
        grid=(_GRID,),
        in_specs=[_pair_spec(_DH2), _pair_spec(16), _full_spec(1, _D),
                  _full_spec(_D, _D), _pair_spec(16)],
        out_specs=_pair_spec(_DH2),
        out_shape=jax.ShapeDtypeStruct((_NC, _N, _DH2), jnp.float32),
    )(p1, dpi, b1r, W2, dpo)

    p2 = _msg_kernel(h2s, srcdst, z64)

    out = pl.pallas_call(
        _tc_head_body,
        grid=(_GRID,),
        in_specs=[_pair_spec(_DH2), _pair_spec(16), _full_spec(1, _D),
                  _full_spec(_D, _D), _full_spec(1, _D),
                  _full_spec(_D, 1), _full_spec(1, 1)],
        out_specs=pl.BlockSpec((_RB, 1), lambda i: (i, 0)),
        out_shape=jax.ShapeDtypeStruct((_N, 1), jnp.float32),
    )(p2, dpi, b2r, wd_p, bd_p, wo_p, bo_p)

    return out


# speedup vs baseline: 1.3671x; 1.1119x over previous
import functools

import jax
import jax.numpy as jnp
from jax import lax
from jax.experimental import pallas as pl
from jax.experimental.pallas import tpu as pltpu
from jax.experimental.pallas import tpu_sc as plsc

_N = 10000
_E = 320000
_D = 128
_DH = 100

_NB = 128
_NC = 2
_NS = 16
_NW = _NC * _NS
_T = 80
_TS = 160
_EPAD = _NS * _TS * _NB
_DH2 = 64
_NPAD = 10112
_RPT = _NPAD // _NS

_RB = 1000
_GRID = _N // _RB

_BUF = 5
_ALD = 3

_sc_mesh = plsc.VectorSubcoreMesh(core_axis_name="c", subcore_axis_name="s")


@functools.partial(
    pl.kernel,
    out_type=(
        jax.ShapeDtypeStruct((_NC, _NPAD, 16), jnp.float32),
        jax.ShapeDtypeStruct((_NC, _NPAD, 16), jnp.float32),
    ),
    mesh=_sc_mesh,
    scratch_types=[
        pltpu.VMEM((_T, _NB), jnp.int32),
        pltpu.VMEM((_T, _NB), jnp.int32),
        pltpu.VMEM((_NB, 16), jnp.float32),
        pltpu.VMEM_SHARED((_NPAD, 16), jnp.float32),
        pltpu.VMEM_SHARED((_NPAD, 16), jnp.float32),
        pltpu.SemaphoreType.DMA,
    ],
)
def _deg_kernel(srcw, dstw, ones_hbm, z16_hbm, dpo, dpi,
                src_v, dst_v, ones_v, sh_o, sh_i, sem):
    cid = lax.axis_index("c")
    sid = lax.axis_index("s")
    r0 = sid * _RPT
    pltpu.sync_copy(z16_hbm.at[pl.ds(r0, _RPT)], sh_o.at[pl.ds(r0, _RPT)])
    pltpu.sync_copy(z16_hbm.at[pl.ds(r0, _RPT)], sh_i.at[pl.ds(r0, _RPT)])
    pltpu.sync_copy(ones_hbm, ones_v)
    pltpu.sync_copy(srcw.at[sid, pl.ds(cid * _T, _T)], src_v)
    pltpu.sync_copy(dstw.at[sid, pl.ds(cid * _T, _T)], dst_v)
    plsc.subcore_barrier()

    def body(j, carry):
        pltpu.async_copy(ones_v, sh_o.at[src_v.at[j]], sem, add=True)
        pltpu.async_copy(ones_v, sh_i.at[dst_v.at[j]], sem, add=True)
        return carry

    lax.fori_loop(0, _T, body, 0)

    def drain(j, carry):
        pltpu.make_async_copy(ones_v, sh_o.at[src_v.at[0]], sem).wait()
        return carry

    lax.fori_loop(0, 2 * _T, drain, 0)
    plsc.subcore_barrier()
    pltpu.sync_copy(sh_o.at[pl.ds(r0, _RPT)], dpo.at[cid, pl.ds(r0, _RPT)])
    pltpu.sync_copy(sh_i.at[pl.ds(r0, _RPT)], dpi.at[cid, pl.ds(r0, _RPT)])


@functools.partial(
    pl.kernel,
    out_type=jax.ShapeDtypeStruct((_NPAD, _D), jnp.float32),
    mesh=_sc_mesh,
    scratch_types=[
        pltpu.VMEM((_BUF, _NB), jnp.int32),
        pltpu.VMEM((_BUF, _NB), jnp.int32),
        pltpu.VMEM((_BUF, _NB, _DH2), jnp.float32),
        pltpu.VMEM_SHARED((_NPAD, _DH2), jnp.float32),
        pltpu.VMEM_SHARED((_N, _DH2), jnp.float32),
    ] + [pltpu.SemaphoreType.DMA] * _BUF,
    compiler_params=pltpu.CompilerParams(use_tc_tiling_on_sc=False),
)
def _msg_kernel(h_hbm, srcw, dstw, z64_hbm, out_hbm,
                idxs_v, idxd_v, rows_v, agg_sh, h_sh, *sems):
    cid = lax.axis_index("c")
    sid = lax.axis_index("s")
    c0 = cid * _DH2
    r0 = sid * _RPT
    pltpu.sync_copy(z64_hbm.at[pl.ds(r0, _RPT)], agg_sh.at[pl.ds(r0, _RPT)])
    nrow = 624
    h0 = sid * nrow
    pltpu.sync_copy(h_hbm.at[pl.ds(h0, nrow), pl.ds(c0, _DH2)],
                    h_sh.at[pl.ds(h0, nrow)])

    @pl.when(sid == 0)
    def _():
        pltpu.sync_copy(h_hbm.at[pl.ds(_NS * nrow, _N - _NS * nrow),
                                 pl.ds(c0, _DH2)],
                        h_sh.at[pl.ds(_NS * nrow, _N - _NS * nrow)])

    hh = h_sh
    sds = srcw.at[sid]
    sdd = dstw.at[sid]
    plsc.subcore_barrier()

    def i_issue(j, b):
        pltpu.async_copy(sds.at[j], idxs_v.at[b], sems[b])
        pltpu.async_copy(sdd.at[j], idxd_v.at[b], sems[b])

    def i_wait(j, b):
        pltpu.make_async_copy(sds.at[j], idxs_v.at[b], sems[b]).wait()
        pltpu.make_async_copy(sdd.at[j], idxd_v.at[b], sems[b]).wait()

    def g_issue(j, b):
        pltpu.async_copy(hh.at[idxs_v.at[b]], rows_v.at[b], sems[b])

    def g_wait(j, b):
        pltpu.make_async_copy(hh.at[idxs_v.at[b]], rows_v.at[b],
                              sems[b]).wait()

    def s_issue(j, b):
        pltpu.async_copy(rows_v.at[b], agg_sh.at[idxd_v.at[b]], sems[b],
                         add=True)

    def s_wait(j, b):
        pltpu.make_async_copy(rows_v.at[b], agg_sh.at[idxd_v.at[b]],
                              sems[b]).wait()

    for j in range(_ALD + 1):
        i_issue(j, j)
    for j in range(_ALD):
        i_wait(j, j)
        g_issue(j, j)

    def body(k, carry):
        for b in range(_BUF):
            j = k * _BUF + b
            g_wait(j, b)
            s_issue(j, b)
            b2 = (b + _ALD + 1) % _BUF

            @pl.when(j + _ALD + 1 - _BUF >= 0)
            def _():
                s_wait(j + _ALD + 1 - _BUF, b2)

            @pl.when(j + _ALD + 1 < _TS)
            def _():
                i_issue(j + _ALD + 1, b2)

            b1 = (b + _ALD) % _BUF

            @pl.when(j + _ALD < _TS)
            def _():
                i_wait(j + _ALD, b1)
                g_issue(j + _ALD, b1)
        return carry

    lax.fori_loop(0, _TS // _BUF, body, 0)
    for j in range(_TS - (_BUF - _ALD - 1), _TS):
        s_wait(j, j % _BUF)
    plsc.subcore_barrier()
    pltpu.sync_copy(agg_sh.at[pl.ds(r0, _RPT)],
                    out_hbm.at[pl.ds(r0, _RPT), pl.ds(c0, _DH2)])


def _norm_from(deg):
    return jnp.where(deg > 0.0, lax.rsqrt(jnp.maximum(deg, 1.0)), 0.0)


def _swish(v):
    return v * jax.nn.sigmoid(v)


def _tc_mm_body(x_ref, w_ref, o_ref):
    o_ref[...] = jnp.dot(x_ref[...], w_ref[...],
                         preferred_element_type=jnp.float32)


def _tc_scale_body(h_ref, d_ref, o_ref):
    deg = d_ref[0, :, 0:1] + d_ref[1, :, 0:1]
    o_ref[...] = h_ref[...] * _norm_from(deg)


def _tc_mid_body(p_ref, di_ref, b_ref, w_ref, do_ref, o_ref):
    deg_in = di_ref[0, :, 0:1] + di_ref[1, :, 0:1]
    agg = p_ref[...] * _norm_from(deg_in)
    h = _swish(agg + b_ref[...])
    deg_out = do_ref[0, :, 0:1] + do_ref[1, :, 0:1]
    o_ref[...] = jnp.dot(h, w_ref[...],
                         preferred_element_type=jnp.float32) * _norm_from(deg_out)


def _tc_head_body(p_ref, di_ref, b_ref, wd_ref, bd_ref, wo_ref, bo_ref, o_ref):
    deg_in = di_ref[0, :, 0:1] + di_ref[1, :, 0:1]
    agg = p_ref[...] * _norm_from(deg_in)
    h = _swish(agg + b_ref[...])
    d = _swish(jnp.dot(h, wd_ref[...], preferred_element_type=jnp.float32)
               + bd_ref[...])
    logit = jnp.dot(d, wo_ref[...], preferred_element_type=jnp.float32) \
        + bo_ref[...]
    o_ref[...] = jax.nn.sigmoid(logit)


def _row_spec(cols):
    return pl.BlockSpec((_RB, cols), lambda i: (i, 0))


def _pair_spec(cols):
    return pl.BlockSpec((2, _RB, cols), lambda i: (0, i, 0))


def _full_spec(rows, cols):
    return pl.BlockSpec((rows, cols), lambda i: (0, 0))


def kernel(x, edge_index, W1, b1, W2, b2, Wd, bd, Wo, bo):
    src = edge_index[0]
    dst = edge_index[1]
    pad_e = _EPAD - _E
    srcw = jnp.concatenate(
        [src, jnp.zeros((pad_e,), jnp.int32)]).reshape(_NS, _TS, _NB)
    dstw = jnp.concatenate(
        [dst, jnp.full((pad_e,), _N, jnp.int32)]).reshape(_NS, _TS, _NB)
    ones2d = jnp.ones((_NB, 16), jnp.float32)
    z16 = jnp.zeros((_NPAD, 16), jnp.float32)
    z64 = jnp.zeros((_NPAD, _DH2), jnp.float32)

    dpo, dpi = _deg_kernel(srcw, dstw, ones2d, z16)

    b1r = b1.reshape(1, -1)
    b2r = b2.reshape(1, -1)
    wd_p = jnp.zeros((_D, _D), jnp.float32).at[:, :_DH].set(Wd)
    bd_p = jnp.zeros((1, _D), jnp.float32).at[0, :_DH].set(bd)
    wo_p = jnp.zeros((_D, 1), jnp.float32).at[:_DH, :].set(Wo)
    bo_p = bo.reshape(1, 1)

    xw = pl.pallas_call(
        _tc_mm_body,
        grid=(_GRID,),
        in_specs=[_row_spec(_D), _full_spec(_D, _D)],
        out_specs=_row_spec(_D),
        out_shape=jax.ShapeDtypeStruct((_N, _D), jnp.float32),
    )(x, W1)

    h1s = pl.pallas_call(
        _tc_scale_body,
        grid=(_GRID,),
        in_specs=[_row_spec(_D), _pair_spec(16)],
        out_specs=_row_spec(_D),
        out_shape=jax.ShapeDtypeStruct((_N, _D), jnp.float32),
    )(xw, dpo)

    p1 = _msg_kernel(h1s, srcw, dstw, z64)

    h2s = pl.pallas_call(
        _tc_mid_body,
        grid=(_GRID,),
        in_specs=[_row_spec(_D), _pair_spec(16), _full_spec(1, _D),
                  _full_spec(_D, _D), _pair_spec(16)],
        out_specs=_row_spec(_D),
        out_shape=jax.ShapeDtypeStruct((_N, _D), jnp.float32),
    )(p1, dpi, b1r, W2, dpo)

    p2 = _msg_kernel(h2s, srcw, dstw, z64)

    out = pl.pallas_call(
        _tc_head_body,
        grid=(_GRID,),
        in_specs=[_row_spec(_D), _pair_spec(16), _full_spec(1, _D),
                  _full_spec(_D, _D), _full_spec(1, _D),
                  _full_spec(_D, 1), _full_spec(1, 1)],
        out_specs=pl.BlockSpec((_RB, 1), lambda i: (i, 0)),
        out_shape=jax.ShapeDtypeStruct((_N, 1), jnp.float32),
    )(p2, dpi, b2r, wd_p, bd_p, wo_p, bo_p)

    return out


# speedup vs baseline: 1.3694x; 1.0017x over previous
import functools

import jax
import jax.numpy as jnp
from jax import lax
from jax.experimental import pallas as pl
from jax.experimental.pallas import tpu as pltpu
from jax.experimental.pallas import tpu_sc as plsc

_N = 10000
_E = 320000
_D = 128
_DH = 100

_NB = 128
_NC = 2
_NS = 16
_NW = _NC * _NS
_T = 80
_TS = 160
_EPAD = _NS * _TS * _NB
_DH2 = 64
_NPAD = 10112
_RPT = _NPAD // _NS

_RB = 1000
_GRID = _N // _RB

_BUF = 5
_ALD = 2

_sc_mesh = plsc.VectorSubcoreMesh(core_axis_name="c", subcore_axis_name="s")


@functools.partial(
    pl.kernel,
    out_type=(
        jax.ShapeDtypeStruct((_NC, _NPAD, 16), jnp.float32),
        jax.ShapeDtypeStruct((_NC, _NPAD, 16), jnp.float32),
    ),
    mesh=_sc_mesh,
    scratch_types=[
        pltpu.VMEM((_T, _NB), jnp.int32),
        pltpu.VMEM((_T, _NB), jnp.int32),
        pltpu.VMEM((_NB, 16), jnp.float32),
        pltpu.VMEM_SHARED((_NPAD, 16), jnp.float32),
        pltpu.VMEM_SHARED((_NPAD, 16), jnp.float32),
        pltpu.SemaphoreType.DMA,
    ],
)
def _deg_kernel(srcw, dstw, ones_hbm, z16_hbm, dpo, dpi,
                src_v, dst_v, ones_v, sh_o, sh_i, sem):
    cid = lax.axis_index("c")
    sid = lax.axis_index("s")
    r0 = sid * _RPT
    pltpu.sync_copy(z16_hbm.at[pl.ds(r0, _RPT)], sh_o.at[pl.ds(r0, _RPT)])
    pltpu.sync_copy(z16_hbm.at[pl.ds(r0, _RPT)], sh_i.at[pl.ds(r0, _RPT)])
    pltpu.sync_copy(ones_hbm, ones_v)
    pltpu.sync_copy(srcw.at[sid, pl.ds(cid * _T, _T)], src_v)
    pltpu.sync_copy(dstw.at[sid, pl.ds(cid * _T, _T)], dst_v)
    plsc.subcore_barrier()

    def body(j, carry):
        pltpu.async_copy(ones_v, sh_o.at[src_v.at[j]], sem, add=True)
        pltpu.async_copy(ones_v, sh_i.at[dst_v.at[j]], sem, add=True)
        return carry

    lax.fori_loop(0, _T, body, 0)

    def drain(j, carry):
        pltpu.make_async_copy(ones_v, sh_o.at[src_v.at[0]], sem).wait()
        return carry

    lax.fori_loop(0, 2 * _T, drain, 0)
    plsc.subcore_barrier()
    pltpu.sync_copy(sh_o.at[pl.ds(r0, _RPT)], dpo.at[cid, pl.ds(r0, _RPT)])
    pltpu.sync_copy(sh_i.at[pl.ds(r0, _RPT)], dpi.at[cid, pl.ds(r0, _RPT)])


@functools.partial(
    pl.kernel,
    out_type=jax.ShapeDtypeStruct((_NPAD, _D), jnp.float32),
    mesh=_sc_mesh,
    scratch_types=[
        pltpu.VMEM((_BUF, _NB), jnp.int32),
        pltpu.VMEM((_BUF, _NB), jnp.int32),
        pltpu.VMEM((_BUF, _NB, _DH2), jnp.float32),
        pltpu.VMEM_SHARED((_NPAD, _DH2), jnp.float32),
        pltpu.VMEM_SHARED((_N, _DH2), jnp.float32),
    ] + [pltpu.SemaphoreType.DMA] * _BUF,
    compiler_params=pltpu.CompilerParams(use_tc_tiling_on_sc=False),
)
def _msg_kernel(h_hbm, srcw, dstw, z64_hbm, out_hbm,
                idxs_v, idxd_v, rows_v, agg_sh, h_sh, *sems):
    cid = lax.axis_index("c")
    sid = lax.axis_index("s")
    c0 = cid * _DH2
    r0 = sid * _RPT
    pltpu.sync_copy(z64_hbm.at[pl.ds(r0, _RPT)], agg_sh.at[pl.ds(r0, _RPT)])
    nrow = 624
    h0 = sid * nrow
    pltpu.sync_copy(h_hbm.at[pl.ds(h0, nrow), pl.ds(c0, _DH2)],
                    h_sh.at[pl.ds(h0, nrow)])

    @pl.when(sid == 0)
    def _():
        pltpu.sync_copy(h_hbm.at[pl.ds(_NS * nrow, _N - _NS * nrow),
                                 pl.ds(c0, _DH2)],
                        h_sh.at[pl.ds(_NS * nrow, _N - _NS * nrow)])

    hh = h_sh
    sds = srcw.at[sid]
    sdd = dstw.at[sid]
    plsc.subcore_barrier()

    def i_issue(j, b):
        pltpu.async_copy(sds.at[j], idxs_v.at[b], sems[b])
        pltpu.async_copy(sdd.at[j], idxd_v.at[b], sems[b])

    def i_wait(j, b):
        pltpu.make_async_copy(sds.at[j], idxs_v.at[b], sems[b]).wait()
        pltpu.make_async_copy(sdd.at[j], idxd_v.at[b], sems[b]).wait()

    def g_issue(j, b):
        pltpu.async_copy(hh.at[idxs_v.at[b]], rows_v.at[b], sems[b])

    def g_wait(j, b):
        pltpu.make_async_copy(hh.at[idxs_v.at[b]], rows_v.at[b],
                              sems[b]).wait()

    def s_issue(j, b):
        pltpu.async_copy(rows_v.at[b], agg_sh.at[idxd_v.at[b]], sems[b],
                         add=True)

    def s_wait(j, b):
        pltpu.make_async_copy(rows_v.at[b], agg_sh.at[idxd_v.at[b]],
                              sems[b]).wait()

    for j in range(_ALD + 1):
        i_issue(j, j)
    for j in range(_ALD):
        i_wait(j, j)
        g_issue(j, j)

    def body(k, carry):
        for b in range(_BUF):
            j = k * _BUF + b
            g_wait(j, b)
            s_issue(j, b)
            b2 = (b + _ALD + 1) % _BUF

            @pl.when(j + _ALD + 1 - _BUF >= 0)
            def _():
                s_wait(j + _ALD + 1 - _BUF, b2)

            @pl.when(j + _ALD + 1 < _TS)
            def _():
                i_issue(j + _ALD + 1, b2)

            b1 = (b + _ALD) % _BUF

            @pl.when(j + _ALD < _TS)
            def _():
                i_wait(j + _ALD, b1)
                g_issue(j + _ALD, b1)
        return carry

    lax.fori_loop(0, _TS // _BUF, body, 0)
    for j in range(_TS - (_BUF - _ALD - 1), _TS):
        s_wait(j, j % _BUF)
    plsc.subcore_barrier()
    pltpu.sync_copy(agg_sh.at[pl.ds(r0, _RPT)],
                    out_hbm.at[pl.ds(r0, _RPT), pl.ds(c0, _DH2)])


def _norm_from(deg):
    return jnp.where(deg > 0.0, lax.rsqrt(jnp.maximum(deg, 1.0)), 0.0)


def _swish(v):
    return v * jax.nn.sigmoid(v)


def _tc_mm_body(x_ref, w_ref, o_ref):
    o_ref[...] = jnp.dot(x_ref[...], w_ref[...],
                         preferred_element_type=jnp.float32)


def _tc_scale_body(h_ref, d_ref, o_ref):
    deg = d_ref[0, :, 0:1] + d_ref[1, :, 0:1]
    o_ref[...] = h_ref[...] * _norm_from(deg)


def _tc_mid_body(p_ref, di_ref, b_ref, w_ref, do_ref, o_ref):
    deg_in = di_ref[0, :, 0:1] + di_ref[1, :, 0:1]
    agg = p_ref[...] * _norm_from(deg_in)
    h = _swish(agg + b_ref[...])
    deg_out = do_ref[0, :, 0:1] + do_ref[1, :, 0:1]
    o_ref[...] = jnp.dot(h, w_ref[...],
                         preferred_element_type=jnp.float32) * _norm_from(deg_out)


def _tc_head_body(p_ref, di_ref, b_ref, wd_ref, bd_ref, wo_ref, bo_ref, o_ref):
    deg_in = di_ref[0, :, 0:1] + di_ref[1, :, 0:1]
    agg = p_ref[...] * _norm_from(deg_in)
    h = _swish(agg + b_ref[...])
    d = _swish(jnp.dot(h, wd_ref[...], preferred_element_type=jnp.float32)
               + bd_ref[...])
    logit = jnp.dot(d, wo_ref[...], preferred_element_type=jnp.float32) \
        + bo_ref[...]
    o_ref[...] = jax.nn.sigmoid(logit)


def _row_spec(cols):
    return pl.BlockSpec((_RB, cols), lambda i: (i, 0))


def _pair_spec(cols):
    return pl.BlockSpec((2, _RB, cols), lambda i: (0, i, 0))


def _full_spec(rows, cols):
    return pl.BlockSpec((rows, cols), lambda i: (0, 0))


def kernel(x, edge_index, W1, b1, W2, b2, Wd, bd, Wo, bo):
    src = edge_index[0]
    dst = edge_index[1]
    pad_e = _EPAD - _E
    srcw = jnp.concatenate(
        [src, jnp.zeros((pad_e,), jnp.int32)]).reshape(_NS, _TS, _NB)
    dstw = jnp.concatenate(
        [dst, jnp.full((pad_e,), _N, jnp.int32)]).reshape(_NS, _TS, _NB)
    ones2d = jnp.ones((_NB, 16), jnp.float32)
    z16 = jnp.zeros((_NPAD, 16), jnp.float32)
    z64 = jnp.zeros((_NPAD, _DH2), jnp.float32)

    dpo, dpi = _deg_kernel(srcw, dstw, ones2d, z16)

    b1r = b1.reshape(1, -1)
    b2r = b2.reshape(1, -1)
    wd_p = jnp.zeros((_D, _D), jnp.float32).at[:, :_DH].set(Wd)
    bd_p = jnp.zeros((1, _D), jnp.float32).at[0, :_DH].set(bd)
    wo_p = jnp.zeros((_D, 1), jnp.float32).at[:_DH, :].set(Wo)
    bo_p = bo.reshape(1, 1)

    xw = pl.pallas_call(
        _tc_mm_body,
        grid=(_GRID,),
        in_specs=[_row_spec(_D), _full_spec(_D, _D)],
        out_specs=_row_spec(_D),
        out_shape=jax.ShapeDtypeStruct((_N, _D), jnp.float32),
    )(x, W1)

    h1s = pl.pallas_call(
        _tc_scale_body,
        grid=(_GRID,),
        in_specs=[_row_spec(_D), _pair_spec(16)],
        out_specs=_row_spec(_D),
        out_shape=jax.ShapeDtypeStruct((_N, _D), jnp.float32),
    )(xw, dpo)

    p1 = _msg_kernel(h1s, srcw, dstw, z64)

    h2s = pl.pallas_call(
        _tc_mid_body,
        grid=(_GRID,),
        in_specs=[_row_spec(_D), _pair_spec(16), _full_spec(1, _D),
                  _full_spec(_D, _D), _pair_spec(16)],
        out_specs=_row_spec(_D),
        out_shape=jax.ShapeDtypeStruct((_N, _D), jnp.float32),
    )(p1, dpi, b1r, W2, dpo)

    p2 = _msg_kernel(h2s, srcw, dstw, z64)

    out = pl.pallas_call(
        _tc_head_body,
        grid=(_GRID,),
        in_specs=[_row_spec(_D), _pair_spec(16), _full_spec(1, _D),
                  _full_spec(_D, _D), _full_spec(1, _D),
                  _full_spec(_D, 1), _full_spec(1, 1)],
        out_specs=pl.BlockSpec((_RB, 1), lambda i: (i, 0)),
        out_shape=jax.ShapeDtypeStruct((_N, 1), jnp.float32),
    )(p2, dpi, b2r, wd_p, bd_p, wo_p, bo_p)

    return out


# speedup vs baseline: 1.4142x; 1.0327x over previous
import functools

import jax
import jax.numpy as jnp
from jax import lax
from jax.experimental import pallas as pl
from jax.experimental.pallas import tpu as pltpu
from jax.experimental.pallas import tpu_sc as plsc

_N = 10000
_E = 320000
_D = 128
_DH = 100

_NB = 128
_NC = 2
_NS = 16
_NW = _NC * _NS
_T = 80
_TS = 160
_EPAD = _NS * _TS * _NB
_DH2 = 64
_NPAD = 10112
_RPT = _NPAD // _NS

_RB = 1000
_GRID = _N // _RB

_BUF = 5
_ALD = 2

_sc_mesh = plsc.VectorSubcoreMesh(core_axis_name="c", subcore_axis_name="s")


_ND = 16384
_NDR = 128


@functools.partial(
    pl.kernel,
    out_type=(
        jax.ShapeDtypeStruct((_NC, _NDR, _NB), jnp.float32),
        jax.ShapeDtypeStruct((_NC, _NDR, _NB), jnp.float32),
    ),
    mesh=_sc_mesh,
    scratch_types=[
        pltpu.VMEM((_T, _NB), jnp.int32),
        pltpu.VMEM((_T, _NB), jnp.int32),
        pltpu.VMEM((_NDR, _NB), jnp.float32),
        pltpu.VMEM((_NDR, _NB), jnp.float32),
        pltpu.VMEM_SHARED((_NDR, _NB), jnp.float32),
        pltpu.VMEM_SHARED((_NDR, _NB), jnp.float32),
        pltpu.VMEM((1, _NB), jnp.int32),
        pltpu.SemaphoreType.DMA,
    ],
    compiler_params=pltpu.CompilerParams(needs_layout_passes=False),
)
def _deg_kernel(srcw, dstw, riota_hbm, dpo, dpi,
                src_v, dst_v, hist_o, hist_i, sh_o, sh_i, riota, sem):
    cid = lax.axis_index("c")
    sid = lax.axis_index("s")
    z16 = jnp.zeros((16,), jnp.float32)
    ones16 = jnp.ones((16,), jnp.float32)

    def zrow(i, carry):
        r = i >> 3
        c = (i & 7) * 16
        hist_o[r, pl.ds(c, 16)] = z16
        hist_i[r, pl.ds(c, 16)] = z16
        return carry

    lax.fori_loop(0, _NDR * 8, zrow, 0)
    pltpu.sync_copy(hist_o.at[pl.ds(0, 8)], sh_o.at[pl.ds(sid * 8, 8)])
    pltpu.sync_copy(hist_o.at[pl.ds(0, 8)], sh_i.at[pl.ds(sid * 8, 8)])
    pltpu.sync_copy(srcw.at[sid, pl.ds(cid * _T, _T)], src_v)
    pltpu.sync_copy(dstw.at[sid, pl.ds(cid * _T, _T)], dst_v)
    pltpu.sync_copy(riota_hbm, riota)
    plsc.subcore_barrier()

    def body(j, carry):
        for c in range(_NB // 16):
            vs = src_v[j, pl.ds(c * 16, 16)]
            plsc.addupdate_scatter(
                hist_o, [lax.shift_right_logical(vs, 7), vs & 127], ones16)
            vd = dst_v[j, pl.ds(c * 16, 16)]
            plsc.addupdate_scatter(
                hist_i, [lax.shift_right_logical(vd, 7), vd & 127], ones16)
        return carry

    lax.fori_loop(0, _T, body, 0)

    pltpu.async_copy(hist_o, sh_o.at[riota.at[0]], sem, add=True)
    pltpu.async_copy(hist_i, sh_i.at[riota.at[0]], sem, add=True)
    pltpu.make_async_copy(hist_o, sh_o.at[riota.at[0]], sem).wait()
    pltpu.make_async_copy(hist_i, sh_i.at[riota.at[0]], sem).wait()
    plsc.subcore_barrier()
    pltpu.sync_copy(sh_o.at[pl.ds(sid * 8, 8)],
                    dpo.at[cid, pl.ds(sid * 8, 8)])
    pltpu.sync_copy(sh_i.at[pl.ds(sid * 8, 8)],
                    dpi.at[cid, pl.ds(sid * 8, 8)])


@functools.partial(
    pl.kernel,
    out_type=jax.ShapeDtypeStruct((_NPAD, _D), jnp.float32),
    mesh=_sc_mesh,
    scratch_types=[
        pltpu.VMEM((_BUF, _NB), jnp.int32),
        pltpu.VMEM((_BUF, _NB), jnp.int32),
        pltpu.VMEM((_BUF, _NB, _DH2), jnp.float32),
        pltpu.VMEM_SHARED((_NPAD, _DH2), jnp.float32),
        pltpu.VMEM_SHARED((_N, _DH2), jnp.float32),
    ] + [pltpu.SemaphoreType.DMA] * _BUF,
    compiler_params=pltpu.CompilerParams(use_tc_tiling_on_sc=False),
)
def _msg_kernel(h_hbm, srcw, dstw, z64_hbm, out_hbm,
                idxs_v, idxd_v, rows_v, agg_sh, h_sh, *sems):
    cid = lax.axis_index("c")
    sid = lax.axis_index("s")
    c0 = cid * _DH2
    r0 = sid * _RPT
    pltpu.sync_copy(z64_hbm.at[pl.ds(r0, _RPT)], agg_sh.at[pl.ds(r0, _RPT)])
    nrow = 624
    h0 = sid * nrow
    pltpu.sync_copy(h_hbm.at[pl.ds(h0, nrow), pl.ds(c0, _DH2)],
                    h_sh.at[pl.ds(h0, nrow)])

    @pl.when(sid == 0)
    def _():
        pltpu.sync_copy(h_hbm.at[pl.ds(_NS * nrow, _N - _NS * nrow),
                                 pl.ds(c0, _DH2)],
                        h_sh.at[pl.ds(_NS * nrow, _N - _NS * nrow)])

    hh = h_sh
    sds = srcw.at[sid]
    sdd = dstw.at[sid]
    plsc.subcore_barrier()

    def i_issue(j, b):
        pltpu.async_copy(sds.at[j], idxs_v.at[b], sems[b])
        pltpu.async_copy(sdd.at[j], idxd_v.at[b], sems[b])

    def i_wait(j, b):
        pltpu.make_async_copy(sds.at[j], idxs_v.at[b], sems[b]).wait()
        pltpu.make_async_copy(sdd.at[j], idxd_v.at[b], sems[b]).wait()

    def g_issue(j, b):
        pltpu.async_copy(hh.at[idxs_v.at[b]], rows_v.at[b], sems[b])

    def g_wait(j, b):
        pltpu.make_async_copy(hh.at[idxs_v.at[b]], rows_v.at[b],
                              sems[b]).wait()

    def s_issue(j, b):
        pltpu.async_copy(rows_v.at[b], agg_sh.at[idxd_v.at[b]], sems[b],
                         add=True)

    def s_wait(j, b):
        pltpu.make_async_copy(rows_v.at[b], agg_sh.at[idxd_v.at[b]],
                              sems[b]).wait()

    for j in range(_ALD + 1):
        i_issue(j, j)
    for j in range(_ALD):
        i_wait(j, j)
        g_issue(j, j)

    def body(k, carry):
        for b in range(_BUF):
            j = k * _BUF + b
            g_wait(j, b)
            s_issue(j, b)
            b2 = (b + _ALD + 1) % _BUF

            @pl.when(j + _ALD + 1 - _BUF >= 0)
            def _():
                s_wait(j + _ALD + 1 - _BUF, b2)

            @pl.when(j + _ALD + 1 < _TS)
            def _():
                i_issue(j + _ALD + 1, b2)

            b1 = (b + _ALD) % _BUF

            @pl.when(j + _ALD < _TS)
            def _():
                i_wait(j + _ALD, b1)
                g_issue(j + _ALD, b1)
        return carry

    lax.fori_loop(0, _TS // _BUF, body, 0)
    for j in range(_TS - (_BUF - _ALD - 1), _TS):
        s_wait(j, j % _BUF)
    plsc.subcore_barrier()
    pltpu.sync_copy(agg_sh.at[pl.ds(r0, _RPT)],
                    out_hbm.at[pl.ds(r0, _RPT), pl.ds(c0, _DH2)])


def _norm_from(deg):
    return jnp.where(deg > 0.0, lax.rsqrt(jnp.maximum(deg, 1.0)), 0.0)


def _swish(v):
    return v * jax.nn.sigmoid(v)


def _tc_mm_body(x_ref, w_ref, o_ref):
    o_ref[...] = jnp.dot(x_ref[...], w_ref[...],
                         preferred_element_type=jnp.float32)


def _tc_scale_body(h_ref, d_ref, o_ref):
    o_ref[...] = h_ref[...] * _norm_from(d_ref[:, 0:1])


def _tc_mid_body(p_ref, di_ref, b_ref, w_ref, do_ref, o_ref):
    agg = p_ref[...] * _norm_from(di_ref[:, 0:1])
    h = _swish(agg + b_ref[...])
    o_ref[...] = jnp.dot(h, w_ref[...],
                         preferred_element_type=jnp.float32) * _norm_from(do_ref[:, 0:1])


def _tc_head_body(p_ref, di_ref, b_ref, wd_ref, bd_ref, wo_ref, bo_ref, o_ref):
    agg = p_ref[...] * _norm_from(di_ref[:, 0:1])
    h = _swish(agg + b_ref[...])
    d = _swish(jnp.dot(h, wd_ref[...], preferred_element_type=jnp.float32)
               + bd_ref[...])
    logit = jnp.dot(d, wo_ref[...], preferred_element_type=jnp.float32) \
        + bo_ref[...]
    o_ref[...] = jax.nn.sigmoid(logit)


def _row_spec(cols):
    return pl.BlockSpec((_RB, cols), lambda i: (i, 0))


def _pair_spec(cols):
    return pl.BlockSpec((2, _RB, cols), lambda i: (0, i, 0))


def _full_spec(rows, cols):
    return pl.BlockSpec((rows, cols), lambda i: (0, 0))


def kernel(x, edge_index, W1, b1, W2, b2, Wd, bd, Wo, bo):
    src = edge_index[0]
    dst = edge_index[1]
    pad_e = _EPAD - _E
    srcw = jnp.concatenate(
        [src, jnp.zeros((pad_e,), jnp.int32)]).reshape(_NS, _TS, _NB)
    dstw = jnp.concatenate(
        [dst, jnp.full((pad_e,), _N, jnp.int32)]).reshape(_NS, _TS, _NB)
    riota = jnp.arange(_NDR, dtype=jnp.int32).reshape(1, _NB)
    z64 = jnp.zeros((_NPAD, _DH2), jnp.float32)

    dpo, dpi = _deg_kernel(srcw, dstw, riota)
    do_bc = jnp.broadcast_to(
        (dpo[0] + dpo[1]).reshape(_ND, 1), (_ND, 16))
    di_bc = jnp.broadcast_to(
        (dpi[0] + dpi[1]).reshape(_ND, 1), (_ND, 16))

    b1r = b1.reshape(1, -1)
    b2r = b2.reshape(1, -1)
    wd_p = jnp.zeros((_D, _D), jnp.float32).at[:, :_DH].set(Wd)
    bd_p = jnp.zeros((1, _D), jnp.float32).at[0, :_DH].set(bd)
    wo_p = jnp.zeros((_D, 1), jnp.float32).at[:_DH, :].set(Wo)
    bo_p = bo.reshape(1, 1)

    xw = pl.pallas_call(
        _tc_mm_body,
        grid=(_GRID,),
        in_specs=[_row_spec(_D), _full_spec(_D, _D)],
        out_specs=_row_spec(_D),
        out_shape=jax.ShapeDtypeStruct((_N, _D), jnp.float32),
    )(x, W1)

    h1s = pl.pallas_call(
        _tc_scale_body,
        grid=(_GRID,),
        in_specs=[_row_spec(_D), _row_spec(16)],
        out_specs=_row_spec(_D),
        out_shape=jax.ShapeDtypeStruct((_N, _D), jnp.float32),
    )(xw, do_bc)

    p1 = _msg_kernel(h1s, srcw, dstw, z64)

    h2s = pl.pallas_call(
        _tc_mid_body,
        grid=(_GRID,),
        in_specs=[_row_spec(_D), _row_spec(16), _full_spec(1, _D),
                  _full_spec(_D, _D), _row_spec(16)],
        out_specs=_row_spec(_D),
        out_shape=jax.ShapeDtypeStruct((_N, _D), jnp.float32),
    )(p1, di_bc, b1r, W2, do_bc)

    p2 = _msg_kernel(h2s, srcw, dstw, z64)

    out = pl.pallas_call(
        _tc_head_body,
        grid=(_GRID,),
        in_specs=[_row_spec(_D), _row_spec(16), _full_spec(1, _D),
                  _full_spec(_D, _D), _full_spec(1, _D),
                  _full_spec(_D, 1), _full_spec(1, 1)],
        out_specs=pl.BlockSpec((_RB, 1), lambda i: (i, 0)),
        out_shape=jax.ShapeDtypeStruct((_N, 1), jnp.float32),
    )(p2, di_bc, b2r, wd_p, bd_p, wo_p, bo_p)

    return out
